# baseline (speedup 1.0000x reference)
import functools

import jax
import jax.numpy as jnp
from jax import lax
from jax.experimental import pallas as pl
from jax.experimental.pallas import tpu as pltpu
from jax.experimental.pallas import tpu_sc as plsc

HIDDEN = 64
MARGIN = 1.0
LANES = 16
NW = 32
CHUNK = 32
NBUF = 2

_PARAMS = pltpu.CompilerParams(
    needs_layout_passes=False, use_tc_tiling_on_sc=True)
_MESH = dict(core_axis_name="c", subcore_axis_name="s")


def _make_transpose(ent_total):
    n_slab = ent_total // 128
    per_w = (n_slab // NW) & ~1
    extra = n_slab - per_w * NW
    assert extra < NW
    tail = ent_total - n_slab * 128

    @functools.partial(
        pl.kernel,
        mesh=plsc.VectorSubcoreMesh(**_MESH),
        compiler_params=_PARAMS,
        out_type=jax.ShapeDtypeStruct((ent_total // 2, 128), jnp.float32),
        scratch_types=[
            pltpu.VMEM((64, 128), jnp.float32),
            pltpu.VMEM((64, 128), jnp.float32),
            pltpu.VMEM((64, 128), jnp.float32),
            pltpu.VMEM((64, 128), jnp.float32),
            pltpu.VMEM((64, 64), jnp.float32),
            pltpu.SemaphoreType.DMA,
            pltpu.SemaphoreType.DMA,
            pltpu.SemaphoreType.DMA,
            pltpu.SemaphoreType.DMA,
        ],
    )
    def k1(entT, out_hbm, sl0, sl1, ov0, ov1, sltail, si0, si1, so0, so1):
        wid = lax.axis_index("s") * 2 + lax.axis_index("c")
        slabs = [sl0, sl1]
        outs = [ov0, ov1]
        sis = [si0, si1]
        sos = [so0, so1]

        def transpose(p):
            slab = slabs[p]
            ov = outs[p]

            def row4(r4, _):
                for dr in range(4):
                    r = r4 * 4 + dr
                    for half in range(2):
                        lane = jnp.full((LANES,), 2 * r + half, jnp.int32)
                        for k in range(HIDDEN // LANES):
                            d_idx = lax.iota(jnp.int32, LANES) + 16 * k
                            vals = plsc.load_gather(slab, [d_idx, lane])
                            ov[r, pl.ds(half * 64 + 16 * k, LANES)] = vals
                return 0

            lax.fori_loop(0, 16, row4, 0)

        def slab_of(j):
            return wid + NW * j

        for p in range(2):
            pltpu.async_copy(
                entT.at[:, pl.ds(pl.multiple_of(slab_of(p) * 128, 128), 128)], slabs[p], sis[p])

        def body(j, _):
            for p in range(2):
                s = slab_of(2 * j + p)
                s128 = pl.multiple_of(s * 128, 128)
                pltpu.make_async_copy(
                    entT.at[:, pl.ds(s128, 128)], slabs[p], sis[p]).wait()
                transpose(p)
                nxt = s + 2 * NW

                @pl.when(2 * j + p + 2 < per_w)
                def _():
                    pltpu.async_copy(
                        entT.at[:, pl.ds(pl.multiple_of(nxt * 128, 128), 128)],
                        slabs[p], sis[p])

                @pl.when(j > 0)
                def _():
                    pltpu.make_async_copy(
                        outs[p], out_hbm.at[pl.ds(s * 64, 64)], sos[p]).wait()
                pltpu.async_copy(
                    outs[p], out_hbm.at[pl.ds(s * 64, 64)], sos[p])
            return 0

        lax.fori_loop(0, per_w // 2, body, 0)
        for p in range(2):
            pltpu.make_async_copy(
                outs[p], out_hbm.at[pl.ds(0, 64)], sos[p]).wait()

        @pl.when(wid < extra)
        def _():
            s = n_slab - extra + wid
            pltpu.sync_copy(
                entT.at[:, pl.ds(pl.multiple_of(s * 128, 128), 128)], sl0)
            transpose(0)
            pltpu.sync_copy(ov0, out_hbm.at[pl.ds(s * 64, 64)])

        if tail:
            @pl.when(wid == extra)
            def _():
                base = n_slab * 128
                pltpu.sync_copy(entT.at[:, pl.ds(base, tail)], sltail)

                def trow(r, _):
                    for half in range(2):
                        lane = jnp.full((LANES,), 2 * r + half, jnp.int32)
                        for k in range(HIDDEN // LANES):
                            d_idx = lax.iota(jnp.int32, LANES) + 16 * k
                            vals = plsc.load_gather(sltail, [d_idx, lane])
                            ov1[r, pl.ds(half * 64 + 16 * k, LANES)] = vals
                    return 0

                lax.fori_loop(0, tail // 2, trow, 0)
                pltpu.sync_copy(
                    ov1.at[pl.ds(0, tail // 2)],
                    out_hbm.at[pl.ds(base // 2, tail // 2)])

    return k1


def _make_gather_loss(batch):
    rows_w = batch // NW
    n_chunks = rows_w // CHUNK
    assert rows_w % CHUNK == 0

    oidx_t = pltpu.VMEM((rows_w + LANES,), jnp.int32)
    pidx_t = pltpu.VMEM((n_chunks, CHUNK), jnp.int32)
    row_t = pltpu.VMEM((NBUF, CHUNK, 128), jnp.float32)

    @functools.partial(
        pl.kernel,
        mesh=plsc.VectorSubcoreMesh(**_MESH),
        compiler_params=_PARAMS,
        out_type=jax.ShapeDtypeStruct((NW, LANES), jnp.float32),
        scratch_types=[
            oidx_t, oidx_t, oidx_t, oidx_t, oidx_t, oidx_t,
            pidx_t, pidx_t, pidx_t, pidx_t, pidx_t, pidx_t,
            row_t, row_t, row_t, row_t, row_t, row_t, row_t, row_t,
            pltpu.VMEM((1, LANES), jnp.float32),
            pltpu.SemaphoreType.DMA,
            pltpu.SemaphoreType.DMA,
        ],
    )
    def k2(ph_hbm, pt_hbm, pr_hbm, nh_hbm, nt_hbm, nr_hbm,
           ent2, rel2, norm2, out_hbm,
           oph, opt, opr, onh, ont, onr,
           pph, ppt, ppr, pnh, pnt, pnr,
           rph, rpt, rpr, rpn, rnh, rnt, rnr, rnn,
           lossv, sem0, sem1):
        wid = lax.axis_index("s") * 2 + lax.axis_index("c")
        base_w = wid * rows_w
        sems = [sem0, sem1]

        sl = pl.ds(base_w, rows_w)
        for src, dst in ((ph_hbm, oph), (pt_hbm, opt), (pr_hbm, opr),
                         (nh_hbm, onh), (nt_hbm, ont), (nr_hbm, onr)):
            pltpu.sync_copy(src.at[sl], dst.at[pl.ds(0, rows_w)])

        def shift_body(c, _):
            for o, p in ((oph, pph), (opt, ppt), (opr, ppr),
                         (onh, pnh), (ont, pnt), (onr, pnr)):
                for jj in range(CHUNK // LANES):
                    v = o[pl.ds(c * CHUNK + jj * LANES, LANES)]
                    p[c, pl.ds(jj * LANES, LANES)] = lax.shift_right_logical(
                        v, 1)
            return 0

        lax.fori_loop(0, n_chunks, shift_body, 0)

        def fire(c):
            b = c % NBUF
            sem = sems[b]
            return [
                pltpu.async_copy(ent2.at[pph.at[c]], rph.at[b], sem),
                pltpu.async_copy(ent2.at[ppt.at[c]], rpt.at[b], sem),
                pltpu.async_copy(rel2.at[ppr.at[c]], rpr.at[b], sem),
                pltpu.async_copy(norm2.at[ppr.at[c]], rpn.at[b], sem),
                pltpu.async_copy(ent2.at[pnh.at[c]], rnh.at[b], sem),
                pltpu.async_copy(ent2.at[pnt.at[c]], rnt.at[b], sem),
                pltpu.async_copy(rel2.at[pnr.at[c]], rnr.at[b], sem),
                pltpu.async_copy(norm2.at[pnr.at[c]], rnn.at[b], sem),
            ]

        loss = jnp.float32(0.0)
        inflight = {0: fire(0)}
        for c in range(n_chunks):
            if c + 1 < n_chunks:
                inflight[c + 1] = fire(c + 1)
            for cp in inflight.pop(c):
                cp.wait()
            b = c % NBUF

            def row_body(i, acc, b=b, c=c):
                base = c * CHUNK + i

                def off(o):
                    return (o[pl.ds(base, LANES)][0] & 1) * 64

                o_ph, o_pt = off(oph), off(opt)
                o_nh, o_nt = off(onh), off(ont)
                o_pr, o_nr = off(opr), off(onr)

                dot_p = jnp.zeros((LANES,), jnp.float32)
                dot_n = jnp.zeros((LANES,), jnp.float32)
                dp, dn, np_, nn_ = [], [], [], []
                for k in range(HIDDEN // LANES):
                    kk = 16 * k
                    d1 = (rph[b, i, pl.ds(o_ph + kk, LANES)]
                          - rpt[b, i, pl.ds(o_pt + kk, LANES)])
                    n1 = rpn[b, i, pl.ds(o_pr + kk, LANES)]
                    dot_p = dot_p + d1 * n1
                    d2 = (rnh[b, i, pl.ds(o_nh + kk, LANES)]
                          - rnt[b, i, pl.ds(o_nt + kk, LANES)])
                    n2 = rnn[b, i, pl.ds(o_nr + kk, LANES)]
                    dot_n = dot_n + d2 * n2
                    dp.append(d1)
                    dn.append(d2)
                    np_.append(n1)
                    nn_.append(n2)
                sp = jnp.sum(dot_p)
                sn = jnp.sum(dot_n)
                comb = jnp.zeros((LANES,), jnp.float32)
                for k in range(HIDDEN // LANES):
                    kk = 16 * k
                    comb = comb + jnp.abs(
                        dp[k] + rpr[b, i, pl.ds(o_pr + kk, LANES)]
                        - sp * np_[k])
                    comb = comb - jnp.abs(
                        dn[k] + rnr[b, i, pl.ds(o_nr + kk, LANES)]
                        - sn * nn_[k])
                return acc + jnp.maximum(jnp.sum(comb) + MARGIN, 0.0)

            loss = lax.fori_loop(0, CHUNK, row_body, loss)

        li = lax.iota(jnp.int32, LANES)
        lossv[0, :] = jnp.where(li == 0, loss, 0.0)
        pltpu.sync_copy(lossv, out_hbm.at[pl.ds(wid, 1)])

    return k2


def kernel(pos_h, pos_t, pos_r, neg_h, neg_t, neg_r,
           ent_embeddings, rel_embeddings, normal_vector):
    batch = pos_h.shape[0]
    ent_total = ent_embeddings.shape[0]
    ent2 = _make_transpose(ent_total)(ent_embeddings.T)
    rel2 = rel_embeddings.reshape(-1, 128)
    norm2 = normal_vector.reshape(-1, 128)
    partials = _make_gather_loss(batch)(
        pos_h, pos_t, pos_r, neg_h, neg_t, neg_r, ent2, rel2, norm2)
    return jnp.sum(partials)


# speedup vs baseline: 1.0006x; 1.0006x over previous
import functools

import jax
import jax.numpy as jnp
from jax import lax
from jax.experimental import pallas as pl
from jax.experimental.pallas import tpu as pltpu
from jax.experimental.pallas import tpu_sc as plsc

HIDDEN = 64
MARGIN = 1.0
LANES = 16
NW = 32
CHUNK = 32
NBUF = 2

_PARAMS = pltpu.CompilerParams(
    needs_layout_passes=False, use_tc_tiling_on_sc=True)
_MESH = dict(core_axis_name="c", subcore_axis_name="s")


def _make_transpose(ent_total):
    n_slab = ent_total // 128
    per_w = (n_slab // NW) & ~3
    extra = n_slab - per_w * NW
    assert extra < NW
    tail = ent_total - n_slab * 128

    @functools.partial(
        pl.kernel,
        mesh=plsc.VectorSubcoreMesh(**_MESH),
        compiler_params=_PARAMS,
        out_type=jax.ShapeDtypeStruct((ent_total // 2, 128), jnp.float32),
        scratch_types=(
            [pltpu.VMEM((64, 128), jnp.float32)] * 8
            + [pltpu.VMEM((64, 64), jnp.float32)]
            + [pltpu.SemaphoreType.DMA] * 8
        ),
    )
    def k1(entT, out_hbm, sl0, sl1, sl2, sl3, ov0, ov1, ov2, ov3, sltail,
           si0, si1, si2, si3, so0, so1, so2, so3):
        wid = lax.axis_index("s") * 2 + lax.axis_index("c")
        slabs = [sl0, sl1, sl2, sl3]
        outs = [ov0, ov1, ov2, ov3]
        sis = [si0, si1, si2, si3]
        sos = [so0, so1, so2, so3]

        def transpose(p):
            slab = slabs[p]
            ov = outs[p]

            def row4(r4, _):
                for dr in range(4):
                    r = r4 * 4 + dr
                    for half in range(2):
                        lane = jnp.full((LANES,), 2 * r + half, jnp.int32)
                        for k in range(HIDDEN // LANES):
                            d_idx = lax.iota(jnp.int32, LANES) + 16 * k
                            vals = plsc.load_gather(slab, [d_idx, lane])
                            ov[r, pl.ds(half * 64 + 16 * k, LANES)] = vals
                return 0

            lax.fori_loop(0, 16, row4, 0)

        def slab_of(j):
            return wid + NW * j

        DEPTH = 4
        for p in range(DEPTH):
            pltpu.async_copy(
                entT.at[:, pl.ds(pl.multiple_of(slab_of(p) * 128, 128), 128)],
                slabs[p], sis[p])

        def body(j, _):
            for p in range(DEPTH):
                s = slab_of(DEPTH * j + p)
                s128 = pl.multiple_of(s * 128, 128)
                pltpu.make_async_copy(
                    entT.at[:, pl.ds(s128, 128)], slabs[p], sis[p]).wait()
                transpose(p)
                nxt = s + DEPTH * NW

                @pl.when(DEPTH * j + p + DEPTH < per_w)
                def _():
                    pltpu.async_copy(
                        entT.at[:, pl.ds(pl.multiple_of(nxt * 128, 128), 128)],
                        slabs[p], sis[p])

                @pl.when(j > 0)
                def _():
                    pltpu.make_async_copy(
                        outs[p], out_hbm.at[pl.ds(s * 64, 64)], sos[p]).wait()
                pltpu.async_copy(
                    outs[p], out_hbm.at[pl.ds(s * 64, 64)], sos[p])
            return 0

        lax.fori_loop(0, per_w // DEPTH, body, 0)
        for p in range(DEPTH):
            pltpu.make_async_copy(
                outs[p], out_hbm.at[pl.ds(0, 64)], sos[p]).wait()

        @pl.when(wid < extra)
        def _():
            s = n_slab - extra + wid
            pltpu.sync_copy(
                entT.at[:, pl.ds(pl.multiple_of(s * 128, 128), 128)], sl0)
            transpose(0)
            pltpu.sync_copy(ov0, out_hbm.at[pl.ds(s * 64, 64)])

        if tail:
            @pl.when(wid == extra)
            def _():
                base = n_slab * 128
                pltpu.sync_copy(entT.at[:, pl.ds(base, tail)], sltail)

                def trow(r, _):
                    for half in range(2):
                        lane = jnp.full((LANES,), 2 * r + half, jnp.int32)
                        for k in range(HIDDEN // LANES):
                            d_idx = lax.iota(jnp.int32, LANES) + 16 * k
                            vals = plsc.load_gather(sltail, [d_idx, lane])
                            ov1[r, pl.ds(half * 64 + 16 * k, LANES)] = vals
                    return 0

                lax.fori_loop(0, tail // 2, trow, 0)
                pltpu.sync_copy(
                    ov1.at[pl.ds(0, tail // 2)],
                    out_hbm.at[pl.ds(base // 2, tail // 2)])

    return k1


def _make_gather_loss(batch):
    rows_w = batch // NW
    n_chunks = rows_w // CHUNK
    assert rows_w % CHUNK == 0

    oidx_t = pltpu.VMEM((rows_w + LANES,), jnp.int32)
    pidx_t = pltpu.VMEM((n_chunks, CHUNK), jnp.int32)
    row_t = pltpu.VMEM((NBUF, CHUNK, 128), jnp.float32)

    @functools.partial(
        pl.kernel,
        mesh=plsc.VectorSubcoreMesh(**_MESH),
        compiler_params=_PARAMS,
        out_type=jax.ShapeDtypeStruct((NW, LANES), jnp.float32),
        scratch_types=[
            oidx_t, oidx_t, oidx_t, oidx_t, oidx_t, oidx_t,
            pidx_t, pidx_t, pidx_t, pidx_t, pidx_t, pidx_t,
            row_t, row_t, row_t, row_t, row_t, row_t, row_t, row_t,
            pltpu.VMEM((1, LANES), jnp.float32),
            pltpu.SemaphoreType.DMA,
            pltpu.SemaphoreType.DMA,
        ],
    )
    def k2(ph_hbm, pt_hbm, pr_hbm, nh_hbm, nt_hbm, nr_hbm,
           ent2, rel2, norm2, out_hbm,
           oph, opt, opr, onh, ont, onr,
           pph, ppt, ppr, pnh, pnt, pnr,
           rph, rpt, rpr, rpn, rnh, rnt, rnr, rnn,
           lossv, sem0, sem1):
        wid = lax.axis_index("s") * 2 + lax.axis_index("c")
        base_w = wid * rows_w
        sems = [sem0, sem1]

        sl = pl.ds(base_w, rows_w)
        for src, dst in ((ph_hbm, oph), (pt_hbm, opt), (pr_hbm, opr),
                         (nh_hbm, onh), (nt_hbm, ont), (nr_hbm, onr)):
            pltpu.sync_copy(src.at[sl], dst.at[pl.ds(0, rows_w)])

        def shift_body(c, _):
            for o, p in ((oph, pph), (opt, ppt), (opr, ppr),
                         (onh, pnh), (ont, pnt), (onr, pnr)):
                for jj in range(CHUNK // LANES):
                    v = o[pl.ds(c * CHUNK + jj * LANES, LANES)]
                    p[c, pl.ds(jj * LANES, LANES)] = lax.shift_right_logical(
                        v, 1)
            return 0

        lax.fori_loop(0, n_chunks, shift_body, 0)

        def fire(c):
            b = c % NBUF
            sem = sems[b]
            return [
                pltpu.async_copy(ent2.at[pph.at[c]], rph.at[b], sem),
                pltpu.async_copy(ent2.at[ppt.at[c]], rpt.at[b], sem),
                pltpu.async_copy(rel2.at[ppr.at[c]], rpr.at[b], sem),
                pltpu.async_copy(norm2.at[ppr.at[c]], rpn.at[b], sem),
                pltpu.async_copy(ent2.at[pnh.at[c]], rnh.at[b], sem),
                pltpu.async_copy(ent2.at[pnt.at[c]], rnt.at[b], sem),
                pltpu.async_copy(rel2.at[pnr.at[c]], rnr.at[b], sem),
                pltpu.async_copy(norm2.at[pnr.at[c]], rnn.at[b], sem),
            ]

        loss = jnp.float32(0.0)
        inflight = {0: fire(0)}
        for c in range(n_chunks):
            if c + 1 < n_chunks:
                inflight[c + 1] = fire(c + 1)
            for cp in inflight.pop(c):
                cp.wait()
            b = c % NBUF

            def row_body(i, acc, b=b, c=c):
                base = c * CHUNK + i

                def off(o):
                    return (o[pl.ds(base, LANES)][0] & 1) * 64

                o_ph, o_pt = off(oph), off(opt)
                o_nh, o_nt = off(onh), off(ont)
                o_pr, o_nr = off(opr), off(onr)

                dot_p = jnp.zeros((LANES,), jnp.float32)
                dot_n = jnp.zeros((LANES,), jnp.float32)
                dp, dn, np_, nn_ = [], [], [], []
                for k in range(HIDDEN // LANES):
                    kk = 16 * k
                    d1 = (rph[b, i, pl.ds(o_ph + kk, LANES)]
                          - rpt[b, i, pl.ds(o_pt + kk, LANES)])
                    n1 = rpn[b, i, pl.ds(o_pr + kk, LANES)]
                    dot_p = dot_p + d1 * n1
                    d2 = (rnh[b, i, pl.ds(o_nh + kk, LANES)]
                          - rnt[b, i, pl.ds(o_nt + kk, LANES)])
                    n2 = rnn[b, i, pl.ds(o_nr + kk, LANES)]
                    dot_n = dot_n + d2 * n2
                    dp.append(d1)
                    dn.append(d2)
                    np_.append(n1)
                    nn_.append(n2)
                sp = jnp.sum(dot_p)
                sn = jnp.sum(dot_n)
                comb = jnp.zeros((LANES,), jnp.float32)
                for k in range(HIDDEN // LANES):
                    kk = 16 * k
                    comb = comb + jnp.abs(
                        dp[k] + rpr[b, i, pl.ds(o_pr + kk, LANES)]
                        - sp * np_[k])
                    comb = comb - jnp.abs(
                        dn[k] + rnr[b, i, pl.ds(o_nr + kk, LANES)]
                        - sn * nn_[k])
                return acc + jnp.maximum(jnp.sum(comb) + MARGIN, 0.0)

            loss = lax.fori_loop(0, CHUNK, row_body, loss)

        li = lax.iota(jnp.int32, LANES)
        lossv[0, :] = jnp.where(li == 0, loss, 0.0)
        pltpu.sync_copy(lossv, out_hbm.at[pl.ds(wid, 1)])

    return k2


def kernel(pos_h, pos_t, pos_r, neg_h, neg_t, neg_r,
           ent_embeddings, rel_embeddings, normal_vector):
    batch = pos_h.shape[0]
    ent_total = ent_embeddings.shape[0]
    ent2 = _make_transpose(ent_total)(ent_embeddings.T)
    rel2 = rel_embeddings.reshape(-1, 128)
    norm2 = normal_vector.reshape(-1, 128)
    partials = _make_gather_loss(batch)(
        pos_h, pos_t, pos_r, neg_h, neg_t, neg_r, ent2, rel2, norm2)
    return jnp.sum(partials)


# speedup vs baseline: 2.4596x; 2.4580x over previous
import functools

import jax
import jax.numpy as jnp
from jax import lax
from jax.experimental import pallas as pl
from jax.experimental.pallas import tpu as pltpu
from jax.experimental.pallas import tpu_sc as plsc

HIDDEN = 64
MARGIN = 1.0
CHUNK = 64
NBUF = 2
LANES = 16


def _make_sc_kernel(batch):
    num_workers = 32
    rows_per_worker = batch // num_workers
    num_chunks = rows_per_worker // CHUNK
    assert rows_per_worker % CHUNK == 0

    mesh = plsc.VectorSubcoreMesh(core_axis_name="c", subcore_axis_name="s")

    idx_t = pltpu.VMEM((rows_per_worker,), jnp.int32)
    row_t = pltpu.VMEM((NBUF, CHUNK, HIDDEN), jnp.float32)

    @functools.partial(
        pl.kernel,
        mesh=mesh,
        compiler_params=pltpu.CompilerParams(
            needs_layout_passes=False, use_tc_tiling_on_sc=False),
        out_type=jax.ShapeDtypeStruct((num_workers, LANES), jnp.float32),
        scratch_types=[
            idx_t, idx_t, idx_t, idx_t, idx_t, idx_t,
            row_t, row_t, row_t, row_t, row_t, row_t, row_t, row_t,
            pltpu.VMEM((1, LANES), jnp.float32),
            pltpu.SemaphoreType.DMA,
            pltpu.SemaphoreType.DMA,
        ],
    )
    def sc_kernel(ph_hbm, pt_hbm, pr_hbm, nh_hbm, nt_hbm, nr_hbm,
                  ent_hbm, rel_hbm, norm_hbm, out_hbm,
                  iph, ipt, ipr, inh, int_, inr,
                  rph, rpt, rpr, rpn, rnh, rnt, rnr, rnn,
                  lossv, sem0, sem1):
        wid = lax.axis_index("s") * 2 + lax.axis_index("c")
        base_w = wid * rows_per_worker
        sems = [sem0, sem1]

        sl = pl.ds(base_w, rows_per_worker)
        pltpu.sync_copy(ph_hbm.at[sl], iph)
        pltpu.sync_copy(pt_hbm.at[sl], ipt)
        pltpu.sync_copy(pr_hbm.at[sl], ipr)
        pltpu.sync_copy(nh_hbm.at[sl], inh)
        pltpu.sync_copy(nt_hbm.at[sl], int_)
        pltpu.sync_copy(nr_hbm.at[sl], inr)

        def fire(c):
            b = c % NBUF
            sem = sems[b]
            return [
                pltpu.async_copy(ent_hbm.at[iph.at[pl.ds(c * CHUNK, CHUNK)]], rph.at[b], sem),
                pltpu.async_copy(ent_hbm.at[ipt.at[pl.ds(c * CHUNK, CHUNK)]], rpt.at[b], sem),
                pltpu.async_copy(rel_hbm.at[ipr.at[pl.ds(c * CHUNK, CHUNK)]], rpr.at[b], sem),
                pltpu.async_copy(norm_hbm.at[ipr.at[pl.ds(c * CHUNK, CHUNK)]], rpn.at[b], sem),
                pltpu.async_copy(ent_hbm.at[inh.at[pl.ds(c * CHUNK, CHUNK)]], rnh.at[b], sem),
                pltpu.async_copy(ent_hbm.at[int_.at[pl.ds(c * CHUNK, CHUNK)]], rnt.at[b], sem),
                pltpu.async_copy(rel_hbm.at[inr.at[pl.ds(c * CHUNK, CHUNK)]], rnr.at[b], sem),
                pltpu.async_copy(norm_hbm.at[inr.at[pl.ds(c * CHUNK, CHUNK)]], rnn.at[b], sem),
            ]

        loss = jnp.float32(0.0)
        inflight = {0: fire(0)}
        for c in range(num_chunks):
            if c + 1 < num_chunks:
                inflight[c + 1] = fire(c + 1)
            for cp in inflight.pop(c):
                cp.wait()
            b = c % NBUF

            def row_body(i, acc, b=b):
                dot_p = jnp.zeros((LANES,), jnp.float32)
                dot_n = jnp.zeros((LANES,), jnp.float32)
                dp = []
                dn = []
                np_ = []
                nn_ = []
                for k in range(HIDDEN // LANES):
                    ds = pl.ds(k * LANES, LANES)
                    d1 = rph[b, i, ds] - rpt[b, i, ds]
                    n1 = rpn[b, i, ds]
                    dot_p = dot_p + d1 * n1
                    d2 = rnh[b, i, ds] - rnt[b, i, ds]
                    n2 = rnn[b, i, ds]
                    dot_n = dot_n + d2 * n2
                    dp.append(d1)
                    dn.append(d2)
                    np_.append(n1)
                    nn_.append(n2)
                sp = jnp.sum(dot_p)
                sn = jnp.sum(dot_n)
                comb = jnp.zeros((LANES,), jnp.float32)
                for k in range(HIDDEN // LANES):
                    ds = pl.ds(k * LANES, LANES)
                    comb = comb + jnp.abs(dp[k] + rpr[b, i, ds] - sp * np_[k])
                    comb = comb - jnp.abs(dn[k] + rnr[b, i, ds] - sn * nn_[k])
                return acc + jnp.maximum(jnp.sum(comb) + MARGIN, 0.0)

            loss = lax.fori_loop(0, CHUNK, row_body, loss)

        li = lax.iota(jnp.int32, LANES)
        lossv[0, :] = jnp.where(li == 0, loss, 0.0)
        pltpu.sync_copy(lossv, out_hbm.at[pl.ds(wid, 1)])

    return sc_kernel


def kernel(pos_h, pos_t, pos_r, neg_h, neg_t, neg_r,
           ent_embeddings, rel_embeddings, normal_vector):
    batch = pos_h.shape[0]
    sc = _make_sc_kernel(batch)
    partials = sc(pos_h, pos_t, pos_r, neg_h, neg_t, neg_r,
                  ent_embeddings, rel_embeddings, normal_vector)
    return jnp.sum(partials)
